# Initial kernel scaffold; baseline (speedup 1.0000x reference)
import functools

import jax
import jax.numpy as jnp
from jax import lax
from jax.experimental import pallas as pl
from jax.experimental.pallas import tpu as pltpu
from jax.experimental.pallas import tpu_sc as plsc

F32 = jnp.float32
HIGH = lax.Precision.HIGHEST

NC = 2
NS = 16
NW = NC * NS

CH = 125
NBUF = 4


def _mesh():
  return plsc.VectorSubcoreMesh(core_axis_name="c", subcore_axis_name="s")


def _deg_sc(dst3, zeros_n, N, E):
  nrow = E // CH
  tpw = nrow // NW

  def body(dst_hbm, z_hbm, out_hbm, acc, idxb, ones, s0, s1, s2, s3):
    sems = (s0, s1, s2, s3)
    c = lax.axis_index("c")
    s = lax.axis_index("s")
    wid = c * NS + s

    for j in range(8):
      ones[pl.ds(j * 16, 16)] = jnp.ones((16,), F32)

    @pl.when(s < NS - 1)
    def _():
      pltpu.sync_copy(z_hbm.at[pl.ds(s * 624, 624)], acc.at[pl.ds(s * 624, 624)])

    @pl.when(s == NS - 1)
    def _():
      pltpu.sync_copy(z_hbm.at[pl.ds(9360, 640)], acc.at[pl.ds(9360, 640)])

    pltpu.sync_copy(dst_hbm.at[pl.ds(wid * tpw, tpw)], idxb)
    plsc.subcore_barrier()

    def group(g, _):
      t0 = g * NBUF
      hs = []
      for b in range(NBUF):
        hs.append(
            pltpu.async_copy(
                ones.at[pl.ds(0, CH)], acc.at[idxb.at[t0 + b]], sems[b],
                add=True))
      for h in hs:
        h.wait()
      return 0

    lax.fori_loop(0, tpw // NBUF, group, 0)
    plsc.subcore_barrier()

    @pl.when(s < NS - 1)
    def _():
      pltpu.sync_copy(acc.at[pl.ds(s * 624, 624)],
                      out_hbm.at[pl.ds(c * N + s * 624, 624)])

    @pl.when(s == NS - 1)
    def _():
      pltpu.sync_copy(acc.at[pl.ds(9360, 640)],
                      out_hbm.at[pl.ds(c * N + 9360, 640)])

  return pl.kernel(
      body,
      out_type=jax.ShapeDtypeStruct((2 * N,), F32),
      mesh=_mesh(),
      scratch_types=[
          pltpu.VMEM_SHARED((N,), F32),
          pltpu.VMEM((tpw, CH), jnp.int32),
          pltpu.VMEM((128,), F32),
          pltpu.SemaphoreType.DMA,
          pltpu.SemaphoreType.DMA,
          pltpu.SemaphoreType.DMA,
          pltpu.SemaphoreType.DMA,
      ],
  )(dst3, zeros_n)


def _agg_sc(hp, ei3, zeros_nd, N, D, E):
  nrow = E // CH
  tpw = nrow // NW

  def body(hp_hbm, ei_hbm, z_hbm, out_hbm, acc, idxb, r0, r1, r2, r3,
           g0, g1, g2, g3, t0_, t1_, t2_, t3_):
    rows = (r0, r1, r2, r3)
    gsem = (g0, g1, g2, g3)
    ssem = (t0_, t1_, t2_, t3_)
    c = lax.axis_index("c")
    s = lax.axis_index("s")
    wid = c * NS + s

    @pl.when(s < NS - 1)
    def _():
      pltpu.sync_copy(z_hbm.at[pl.ds(s * 624, 624)], acc.at[pl.ds(s * 624, 624)])

    @pl.when(s == NS - 1)
    def _():
      pltpu.sync_copy(z_hbm.at[pl.ds(9360, 640)], acc.at[pl.ds(9360, 640)])

    pltpu.sync_copy(ei_hbm.at[:, pl.ds(wid * tpw, tpw)], idxb)
    plsc.subcore_barrier()

    def group(g, _):
      t0 = g * NBUF
      hs = []
      for b in range(NBUF):
        hs.append(
            pltpu.async_copy(hp_hbm.at[idxb.at[0, t0 + b]], rows[b], gsem[b]))
      ws = []
      for b in range(NBUF):
        hs[b].wait()
        ws.append(
            pltpu.async_copy(rows[b], acc.at[idxb.at[1, t0 + b]], ssem[b],
                             add=True))
      for w in ws:
        w.wait()
      return 0

    lax.fori_loop(0, tpw // NBUF, group, 0)
    plsc.subcore_barrier()

    @pl.when(s < NS - 1)
    def _():
      pltpu.sync_copy(acc.at[pl.ds(s * 624, 624)],
                      out_hbm.at[pl.ds(c * N + s * 624, 624)])

    @pl.when(s == NS - 1)
    def _():
      pltpu.sync_copy(acc.at[pl.ds(9360, 640)],
                      out_hbm.at[pl.ds(c * N + 9360, 640)])

  return pl.kernel(
      body,
      out_type=jax.ShapeDtypeStruct((2 * N, D), F32),
      mesh=_mesh(),
      scratch_types=[
          pltpu.VMEM_SHARED((N, D), F32),
          pltpu.VMEM((2, tpw, CH), jnp.int32),
          pltpu.VMEM((CH, D), F32),
          pltpu.VMEM((CH, D), F32),
          pltpu.VMEM((CH, D), F32),
          pltpu.VMEM((CH, D), F32),
          pltpu.SemaphoreType.DMA,
          pltpu.SemaphoreType.DMA,
          pltpu.SemaphoreType.DMA,
          pltpu.SemaphoreType.DMA,
          pltpu.SemaphoreType.DMA,
          pltpu.SemaphoreType.DMA,
          pltpu.SemaphoreType.DMA,
          pltpu.SemaphoreType.DMA,
      ],
  )(hp, ei3, zeros_nd)


_R = 1000


def _mm1_body(x_ref, w_ref, d0_ref, d1_ref, hp_ref, dis_ref):
  deg = d0_ref[...] + d1_ref[...] + 1.0
  dis = lax.rsqrt(deg)
  h = jnp.dot(x_ref[...], w_ref[...], preferred_element_type=F32,
              precision=HIGH)
  dis_ref[...] = dis
  hp_ref[...] = dis * h


def _mm1(x, w1, d0, d1, N, D):
  return pl.pallas_call(
      _mm1_body,
      grid=(N // _R,),
      in_specs=[
          pl.BlockSpec((_R, D), lambda i: (i, 0)),
          pl.BlockSpec((D, D), lambda i: (0, 0)),
          pl.BlockSpec((_R, 1), lambda i: (i, 0)),
          pl.BlockSpec((_R, 1), lambda i: (i, 0)),
      ],
      out_specs=[
          pl.BlockSpec((_R, D), lambda i: (i, 0)),
          pl.BlockSpec((_R, 1), lambda i: (i, 0)),
      ],
      out_shape=[
          jax.ShapeDtypeStruct((N, D), F32),
          jax.ShapeDtypeStruct((N, 1), F32),
      ],
  )(x, w1, d0, d1)


def _mm2_body(a0_ref, a1_ref, hp_ref, dis_ref, b_ref, w_ref, out_ref):
  dis = dis_ref[...]
  pre = dis * (a0_ref[...] + a1_ref[...] + hp_ref[...]) + b_ref[...]
  x2 = jnp.maximum(pre, 0.0)
  out_ref[...] = dis * jnp.dot(x2, w_ref[...], preferred_element_type=F32,
                               precision=HIGH)


def _mm2(a0, a1, hp, dis, b1, w2, N, D):
  return pl.pallas_call(
      _mm2_body,
      grid=(N // _R,),
      in_specs=[
          pl.BlockSpec((_R, D), lambda i: (i, 0)),
          pl.BlockSpec((_R, D), lambda i: (i, 0)),
          pl.BlockSpec((_R, D), lambda i: (i, 0)),
          pl.BlockSpec((_R, 1), lambda i: (i, 0)),
          pl.BlockSpec((1, D), lambda i: (0, 0)),
          pl.BlockSpec((D, D), lambda i: (0, 0)),
      ],
      out_specs=pl.BlockSpec((_R, D), lambda i: (i, 0)),
      out_shape=jax.ShapeDtypeStruct((N, D), F32),
  )(a0, a1, hp, dis, b1, w2)


def _mm3_body(a0_ref, a1_ref, hp_ref, dis_ref, b_ref, bt_ref, wp_ref, bp_ref,
              out_ref, pool_acc, cnt_acc, *, nblk, G):
  i = pl.program_id(0)
  dis = dis_ref[...]
  pre = dis * (a0_ref[...] + a1_ref[...] + hp_ref[...]) + b_ref[...]
  x3 = jnp.maximum(pre, 0.0)
  st = (bt_ref[...] == lax.broadcasted_iota(jnp.int32, (_R, G), 1)).astype(F32)
  ppart = lax.dot_general(st, x3, (((0,), (0,)), ((), ())),
                          preferred_element_type=F32, precision=HIGH)
  cpart = lax.dot_general(st, jnp.ones((_R, 1), F32), (((0,), (0,)), ((), ())),
                          preferred_element_type=F32, precision=HIGH)

  @pl.when(i == 0)
  def _():
    pool_acc[...] = ppart
    cnt_acc[...] = cpart

  @pl.when(i > 0)
  def _():
    pool_acc[...] += ppart
    cnt_acc[...] += cpart

  @pl.when(i == nblk - 1)
  def _():
    pooled = pool_acc[...] / jnp.maximum(cnt_acc[...], 1.0)
    out_ref[...] = jnp.dot(pooled, wp_ref[...], preferred_element_type=F32,
                           precision=HIGH) + bp_ref[...]


def _mm3(a0, a1, hp, dis, b2, bt, wp, bp, N, D, G):
  nblk = N // _R
  return pl.pallas_call(
      functools.partial(_mm3_body, nblk=nblk, G=G),
      grid=(nblk,),
      in_specs=[
          pl.BlockSpec((_R, D), lambda i: (i, 0)),
          pl.BlockSpec((_R, D), lambda i: (i, 0)),
          pl.BlockSpec((_R, D), lambda i: (i, 0)),
          pl.BlockSpec((_R, 1), lambda i: (i, 0)),
          pl.BlockSpec((1, D), lambda i: (0, 0)),
          pl.BlockSpec((_R, 1), lambda i: (i, 0)),
          pl.BlockSpec((D, D), lambda i: (0, 0)),
          pl.BlockSpec((1, D), lambda i: (0, 0)),
      ],
      out_specs=pl.BlockSpec((G, D), lambda i: (0, 0)),
      out_shape=jax.ShapeDtypeStruct((G, D), F32),
      scratch_shapes=[
          pltpu.VMEM((G, D), F32),
          pltpu.VMEM((G, 1), F32),
      ],
  )(a0, a1, hp, dis, b2, bt, wp, bp)


def kernel(x, edge_index, batch, W1, b1, W2, b2, Wp, bp):
  N, D = x.shape
  E = edge_index.shape[1]
  G = 64

  ei3 = edge_index.reshape(2, E // CH, CH)
  dst3 = edge_index[1].reshape(E // CH, CH)
  zeros_n = jnp.zeros((N,), F32)
  zeros_nd = jnp.zeros((N, D), F32)

  degp = _deg_sc(dst3, zeros_n, N, E)
  d0 = degp[:N].reshape(N, 1)
  d1 = degp[N:].reshape(N, 1)

  h1p, dis = _mm1(x, W1, d0, d1, N, D)

  agg1 = _agg_sc(h1p, ei3, zeros_nd, N, D, E)
  h2p = _mm2(agg1[:N], agg1[N:], h1p, dis, b1.reshape(1, D), W2, N, D)

  agg2 = _agg_sc(h2p, ei3, zeros_nd, N, D, E)
  out = _mm3(agg2[:N], agg2[N:], h2p, dis, b2.reshape(1, D),
             batch.reshape(N, 1), Wp, bp.reshape(1, D), N, D, G)
  return out


# speedup vs baseline: 24.4173x; 24.4173x over previous
import functools

import jax
import jax.numpy as jnp
from jax import lax
from jax.experimental import pallas as pl
from jax.experimental.pallas import tpu as pltpu
from jax.experimental.pallas import tpu_sc as plsc

F32 = jnp.float32
HIGH = lax.Precision.HIGHEST

NC = 2
NS = 16
NW = NC * NS

CH = 80
NBUF = 4


def _mesh():
  return plsc.VectorSubcoreMesh(core_axis_name="c", subcore_axis_name="s",
                                num_cores=NC, num_subcores=NS)


def _deg_sc(dst3, N, E):
  nrow = E // CH
  tpw = nrow // NW

  def body(dst_hbm, out_hbm, acc, idxb, ones, wb, s0, s1, s2, s3):
    sems = (s0, s1, s2, s3)
    c = lax.axis_index("c")
    s = lax.axis_index("s")
    wid = c * NS + s

    for j in range(8):
      ones[pl.ds(j * 16, 16)] = jnp.ones((16,), F32)

    def zb(j, _):
      wb[pl.ds(j * 16, 16)] = jnp.zeros((16,), F32)
      return 0

    lax.fori_loop(0, 40, zb, 0)

    @pl.when(s < NS - 1)
    def _():
      pltpu.sync_copy(wb.at[pl.ds(0, 624)], acc.at[pl.ds(s * 624, 624)])

    @pl.when(s == NS - 1)
    def _():
      pltpu.sync_copy(wb, acc.at[pl.ds(9360, 640)])

    pltpu.sync_copy(dst_hbm.at[pl.ds(wid * tpw, tpw)], idxb)
    plsc.subcore_barrier()

    def group(g, _):
      t0 = g * NBUF
      hs = []
      for b in range(NBUF):
        hs.append(
            pltpu.async_copy(
                ones.at[pl.ds(0, CH)], acc.at[idxb.at[t0 + b]], sems[b],
                add=True))
      for h in hs:
        h.wait()
      return 0

    ngr = tpw // NBUF
    lax.fori_loop(0, ngr, group, 0)
    for r in range(ngr * NBUF, tpw):
      pltpu.async_copy(ones.at[pl.ds(0, CH)], acc.at[idxb.at[r]], sems[0],
                       add=True).wait()
    plsc.subcore_barrier()

    @pl.when(s < NS - 1)
    def _():
      pltpu.sync_copy(acc.at[pl.ds(s * 624, 624)], wb.at[pl.ds(0, 624)])
      pltpu.sync_copy(wb.at[pl.ds(0, 624)],
                      out_hbm.at[pl.ds(c * N + s * 624, 624)])

    @pl.when(s == NS - 1)
    def _():
      pltpu.sync_copy(acc.at[pl.ds(9360, 640)], wb)
      pltpu.sync_copy(wb, out_hbm.at[pl.ds(c * N + 9360, 640)])

  return pl.kernel(
      body,
      out_type=jax.ShapeDtypeStruct((2 * N,), F32),
      mesh=_mesh(),
      scratch_types=[
          pltpu.VMEM_SHARED((N,), F32),
          pltpu.VMEM((tpw, CH), jnp.int32),
          pltpu.VMEM((128,), F32),
          pltpu.VMEM((640,), F32),
          pltpu.SemaphoreType.DMA,
          pltpu.SemaphoreType.DMA,
          pltpu.SemaphoreType.DMA,
          pltpu.SemaphoreType.DMA,
      ],
      compiler_params=pltpu.CompilerParams(use_tc_tiling_on_sc=False),
  )(dst3)


def _agg_sc(hp, src3, dst3, N, D, E):
  nrow = E // CH
  tpw = nrow // NW
  rpt = N // NS

  def body(hp_hbm, src_hbm, dst_hbm, out_hbm, acc, idxs, idxd, r0, r1, r2, r3,
           g0, g1, g2, g3, t0_, t1_, t2_, t3_):
    rows = (r0, r1, r2, r3)
    gsem = (g0, g1, g2, g3)
    ssem = (t0_, t1_, t2_, t3_)
    c = lax.axis_index("c")
    s = lax.axis_index("s")
    wid = c * NS + s

    def zrow(j, _):
      for k in range(8):
        r0[j, pl.ds(k * 16, 16)] = jnp.zeros((16,), F32)
      return 0

    lax.fori_loop(0, CH, zrow, 0)
    nwb = N // 80
    for k in range(8):
      ch = s + NS * k

      @pl.when(ch < nwb)
      def _():
        pltpu.sync_copy(r0.at[pl.ds(0, 80)], acc.at[pl.ds(ch * 80, 80)])

    plsc.subcore_barrier()

    half = 25
    for hstart in range(0, tpw, half):
      pltpu.sync_copy(src_hbm.at[pl.ds(wid * tpw + hstart, half)], idxs)
      pltpu.sync_copy(dst_hbm.at[pl.ds(wid * tpw + hstart, half)], idxd)

      def group(g, _):
        t0 = g * NBUF
        hs = []
        for b in range(NBUF):
          hs.append(
              pltpu.async_copy(hp_hbm.at[idxs.at[t0 + b]], rows[b], gsem[b]))
        ws = []
        for b in range(NBUF):
          hs[b].wait()
          ws.append(
              pltpu.async_copy(rows[b], acc.at[idxd.at[t0 + b]], ssem[b],
                               add=True))
        for w in ws:
          w.wait()
        return 0

      ngr = half // NBUF
      lax.fori_loop(0, ngr, group, 0)
      for r in range(ngr * NBUF, half):
        pltpu.async_copy(hp_hbm.at[idxs.at[r]], rows[0], gsem[0]).wait()
        pltpu.async_copy(rows[0], acc.at[idxd.at[r]], ssem[0],
                         add=True).wait()
    plsc.subcore_barrier()

    for k in range(8):
      ch = s + NS * k

      @pl.when(ch < nwb)
      def _():
        buf = rows[k % NBUF]
        pltpu.sync_copy(acc.at[pl.ds(ch * 80, 80)], buf.at[pl.ds(0, 80)])
        pltpu.sync_copy(buf.at[pl.ds(0, 80)],
                        out_hbm.at[pl.ds(c * N + ch * 80, 80)])

  return pl.kernel(
      body,
      out_type=jax.ShapeDtypeStruct((2 * N, D), F32),
      mesh=_mesh(),
      scratch_types=[
          pltpu.VMEM_SHARED((N, D), F32),
          pltpu.VMEM((25, CH), jnp.int32),
          pltpu.VMEM((25, CH), jnp.int32),
          pltpu.VMEM((CH, D), F32),
          pltpu.VMEM((CH, D), F32),
          pltpu.VMEM((CH, D), F32),
          pltpu.VMEM((CH, D), F32),
          pltpu.SemaphoreType.DMA,
          pltpu.SemaphoreType.DMA,
          pltpu.SemaphoreType.DMA,
          pltpu.SemaphoreType.DMA,
          pltpu.SemaphoreType.DMA,
          pltpu.SemaphoreType.DMA,
          pltpu.SemaphoreType.DMA,
          pltpu.SemaphoreType.DMA,
      ],
      compiler_params=pltpu.CompilerParams(use_tc_tiling_on_sc=False),
  )(hp, src3, dst3)


_R = 1000


def _mm1_body(x_ref, w_ref, d0_ref, d1_ref, hp_ref, dis_ref):
  deg = d0_ref[...] + d1_ref[...] + 1.0
  dis = lax.rsqrt(deg)
  h = jnp.dot(x_ref[...], w_ref[...], preferred_element_type=F32,
              precision=HIGH)
  dis_ref[...] = dis
  hp_ref[...] = dis * h


def _mm1(x, w1, d0, d1, N, D):
  return pl.pallas_call(
      _mm1_body,
      grid=(N // _R,),
      in_specs=[
          pl.BlockSpec((_R, D), lambda i: (i, 0)),
          pl.BlockSpec((D, D), lambda i: (0, 0)),
          pl.BlockSpec((_R, 1), lambda i: (i, 0)),
          pl.BlockSpec((_R, 1), lambda i: (i, 0)),
      ],
      out_specs=[
          pl.BlockSpec((_R, D), lambda i: (i, 0)),
          pl.BlockSpec((_R, 1), lambda i: (i, 0)),
      ],
      out_shape=[
          jax.ShapeDtypeStruct((N, D), F32),
          jax.ShapeDtypeStruct((N, 1), F32),
      ],
  )(x, w1, d0, d1)


def _mm2_body(a0_ref, a1_ref, hp_ref, dis_ref, b_ref, w_ref, out_ref):
  dis = dis_ref[...]
  pre = dis * (a0_ref[...] + a1_ref[...] + hp_ref[...]) + b_ref[...]
  x2 = jnp.maximum(pre, 0.0)
  out_ref[...] = dis * jnp.dot(x2, w_ref[...], preferred_element_type=F32,
                               precision=HIGH)


def _mm2(a0, a1, hp, dis, b1, w2, N, D):
  return pl.pallas_call(
      _mm2_body,
      grid=(N // _R,),
      in_specs=[
          pl.BlockSpec((_R, D), lambda i: (i, 0)),
          pl.BlockSpec((_R, D), lambda i: (i, 0)),
          pl.BlockSpec((_R, D), lambda i: (i, 0)),
          pl.BlockSpec((_R, 1), lambda i: (i, 0)),
          pl.BlockSpec((1, D), lambda i: (0, 0)),
          pl.BlockSpec((D, D), lambda i: (0, 0)),
      ],
      out_specs=pl.BlockSpec((_R, D), lambda i: (i, 0)),
      out_shape=jax.ShapeDtypeStruct((N, D), F32),
  )(a0, a1, hp, dis, b1, w2)


def _mm3_body(a0_ref, a1_ref, hp_ref, dis_ref, b_ref, bt_ref, wp_ref, bp_ref,
              out_ref, pool_acc, cnt_acc, *, nblk, G):
  i = pl.program_id(0)
  dis = dis_ref[...]
  pre = dis * (a0_ref[...] + a1_ref[...] + hp_ref[...]) + b_ref[...]
  x3 = jnp.maximum(pre, 0.0)
  st = (bt_ref[...] == lax.broadcasted_iota(jnp.int32, (_R, G), 1)).astype(F32)
  ppart = lax.dot_general(st, x3, (((0,), (0,)), ((), ())),
                          preferred_element_type=F32, precision=HIGH)
  cpart = lax.dot_general(st, jnp.ones((_R, 1), F32), (((0,), (0,)), ((), ())),
                          preferred_element_type=F32, precision=HIGH)

  @pl.when(i == 0)
  def _():
    pool_acc[...] = ppart
    cnt_acc[...] = cpart

  @pl.when(i > 0)
  def _():
    pool_acc[...] += ppart
    cnt_acc[...] += cpart

  @pl.when(i == nblk - 1)
  def _():
    pooled = pool_acc[...] / jnp.maximum(cnt_acc[...], 1.0)
    out_ref[...] = jnp.dot(pooled, wp_ref[...], preferred_element_type=F32,
                           precision=HIGH) + bp_ref[...]


def _mm3(a0, a1, hp, dis, b2, bt, wp, bp, N, D, G):
  nblk = N // _R
  return pl.pallas_call(
      functools.partial(_mm3_body, nblk=nblk, G=G),
      grid=(nblk,),
      in_specs=[
          pl.BlockSpec((_R, D), lambda i: (i, 0)),
          pl.BlockSpec((_R, D), lambda i: (i, 0)),
          pl.BlockSpec((_R, D), lambda i: (i, 0)),
          pl.BlockSpec((_R, 1), lambda i: (i, 0)),
          pl.BlockSpec((1, D), lambda i: (0, 0)),
          pl.BlockSpec((_R, 1), lambda i: (i, 0)),
          pl.BlockSpec((D, D), lambda i: (0, 0)),
          pl.BlockSpec((1, D), lambda i: (0, 0)),
      ],
      out_specs=pl.BlockSpec((G, D), lambda i: (0, 0)),
      out_shape=jax.ShapeDtypeStruct((G, D), F32),
      scratch_shapes=[
          pltpu.VMEM((G, D), F32),
          pltpu.VMEM((G, 1), F32),
      ],
  )(a0, a1, hp, dis, b2, bt, wp, bp)


def kernel(x, edge_index, batch, W1, b1, W2, b2, Wp, bp):
  N, D = x.shape
  E = edge_index.shape[1]
  G = 64

  src3 = edge_index[0].reshape(E // CH, CH)
  dst3 = edge_index[1].reshape(E // CH, CH)

  degp = _deg_sc(dst3, N, E)
  d0 = degp[:N].reshape(N, 1)
  d1 = degp[N:].reshape(N, 1)

  h1p, dis = _mm1(x, W1, d0, d1, N, D)

  agg1 = _agg_sc(h1p, src3, dst3, N, D, E)
  h2p = _mm2(agg1[:N], agg1[N:], h1p, dis, b1.reshape(1, D), W2, N, D)

  agg2 = _agg_sc(h2p, src3, dst3, N, D, E)
  out = _mm3(agg2[:N], agg2[N:], h2p, dis, b2.reshape(1, D),
             batch.reshape(N, 1), Wp, bp.reshape(1, D), N, D, G)
  return out
